# baseline (speedup 1.0000x reference)
import functools

import jax
import jax.numpy as jnp
from jax import lax
from jax.experimental import pallas as pl
from jax.experimental.pallas import tpu as pltpu
from jax.experimental.pallas import tpu_sc as plsc

N_NODES = 10000
N_EDGES = 320000
D = 128

NC = 2
NS = 16
NW = NC * NS
N_PAD = 10240
ROWS_PER_TILE = N_PAD // NS
EPW = N_EDGES // NW
CHUNK = 128
NCHUNK = EPW // CHUNK
TAIL = EPW - NCHUNK * CHUNK


def _sc_aggregate(x, ei, z2, z1):
    mesh = plsc.VectorSubcoreMesh(core_axis_name="c", subcore_axis_name="s")

    @functools.partial(
        pl.kernel,
        out_type=[
            jax.ShapeDtypeStruct((NC, N_PAD, D), jnp.float32),
            jax.ShapeDtypeStruct((NC, N_PAD), jnp.float32),
        ],
        mesh=mesh,
        scratch_types=[
            pltpu.VMEM((CHUNK,), jnp.int32),
            pltpu.VMEM((CHUNK,), jnp.int32),
            pltpu.VMEM((CHUNK,), jnp.int32),
            pltpu.VMEM((CHUNK,), jnp.int32),
            pltpu.VMEM((TAIL,), jnp.int32),
            pltpu.VMEM((TAIL,), jnp.int32),
            pltpu.VMEM((CHUNK, D), jnp.float32),
            pltpu.VMEM((CHUNK, D), jnp.float32),
            pltpu.VMEM((CHUNK,), jnp.float32),
            pltpu.VMEM_SHARED((N_PAD, D), jnp.float32),
            pltpu.VMEM_SHARED((N_PAD,), jnp.float32),
            pltpu.SemaphoreType.DMA,
            pltpu.SemaphoreType.DMA,
            pltpu.SemaphoreType.DMA,
            pltpu.SemaphoreType.DMA,
        ],
    )
    def agg(x_hbm, ei_hbm, z2_hbm, z1_hbm, acc_out, deg_out,
            src_a, src_b, dst_a, dst_b, tsrc, tdst, rows_a, rows_b, ones_v,
            acc_s, deg_s, sem_a, sem_b, sem_ia, sem_ib):
        c = lax.axis_index("c")
        s = lax.axis_index("s")
        wid = s * NC + c
        ebase = wid * EPW
        rbase = s * ROWS_PER_TILE

        def idx_load(k, src_v, dst_v, sem):
            pltpu.async_copy(
                ei_hbm.at[pl.ds(ebase + k * CHUNK, CHUNK)], src_v, sem)
            pltpu.async_copy(
                ei_hbm.at[pl.ds(N_EDGES + ebase + k * CHUNK, CHUNK)],
                dst_v, sem)

        def idx_wait(k, src_v, dst_v, sem):
            pltpu.make_async_copy(
                ei_hbm.at[pl.ds(ebase + k * CHUNK, CHUNK)], src_v,
                sem).wait()
            pltpu.make_async_copy(
                ei_hbm.at[pl.ds(N_EDGES + ebase + k * CHUNK, CHUNK)],
                dst_v, sem).wait()

        def gather(src_v, buf, sem):
            pltpu.async_copy(x_hbm.at[src_v], buf, sem)

        def gwait(src_v, buf, sem):
            pltpu.make_async_copy(x_hbm.at[src_v], buf, sem).wait()

        def flush(dst_v, buf):
            pltpu.sync_copy(buf, acc_s.at[dst_v], add=True)
            pltpu.sync_copy(ones_v, deg_s.at[dst_v], add=True)

        idx_load(0, src_a, dst_a, sem_ia)
        pltpu.sync_copy(z2_hbm, rows_a)
        for j in range(ROWS_PER_TILE // CHUNK):
            pltpu.sync_copy(rows_a, acc_s.at[pl.ds(rbase + j * CHUNK, CHUNK)])
        pltpu.sync_copy(z1_hbm.at[pl.ds(rbase, ROWS_PER_TILE)],
                        deg_s.at[pl.ds(rbase, ROWS_PER_TILE)])
        for j in range(CHUNK // 16):
            ones_v[pl.ds(j * 16, 16)] = jnp.ones((16,), jnp.float32)
        plsc.subcore_barrier()

        idx_wait(0, src_a, dst_a, sem_ia)
        gather(src_a, rows_a, sem_a)
        idx_load(1, src_b, dst_b, sem_ib)

        def body(i, carry):
            k = 2 * i
            idx_wait(k + 1, src_b, dst_b, sem_ib)
            gather(src_b, rows_b, sem_b)
            gwait(src_a, rows_a, sem_a)
            flush(dst_a, rows_a)
            idx_load(k + 2, src_a, dst_a, sem_ia)
            gwait(src_b, rows_b, sem_b)
            flush(dst_b, rows_b)
            idx_wait(k + 2, src_a, dst_a, sem_ia)
            gather(src_a, rows_a, sem_a)
            idx_load(k + 3, src_b, dst_b, sem_ib)
            return carry

        lax.fori_loop(0, (NCHUNK - 2) // 2, body, 0)
        idx_wait(NCHUNK - 1, src_b, dst_b, sem_ib)
        gather(src_b, rows_b, sem_b)
        pltpu.sync_copy(
            ei_hbm.at[pl.ds(ebase + NCHUNK * CHUNK, TAIL)], tsrc)
        pltpu.sync_copy(
            ei_hbm.at[pl.ds(N_EDGES + ebase + NCHUNK * CHUNK, TAIL)], tdst)
        gwait(src_a, rows_a, sem_a)
        flush(dst_a, rows_a)
        pltpu.async_copy(x_hbm.at[tsrc], rows_a.at[pl.ds(0, TAIL)],
                         sem_a).wait()
        gwait(src_b, rows_b, sem_b)
        flush(dst_b, rows_b)
        pltpu.sync_copy(rows_a.at[pl.ds(0, TAIL)], acc_s.at[tdst], add=True)
        pltpu.sync_copy(ones_v.at[pl.ds(0, TAIL)], deg_s.at[tdst], add=True)
        plsc.subcore_barrier()

        pltpu.sync_copy(acc_s.at[pl.ds(rbase, ROWS_PER_TILE)],
                        acc_out.at[c, pl.ds(rbase, ROWS_PER_TILE)])
        pltpu.sync_copy(deg_s.at[pl.ds(rbase, ROWS_PER_TILE)],
                        deg_out.at[c, pl.ds(rbase, ROWS_PER_TILE)])

    return agg(x, ei, z2, z1)


def _tc_epilogue(x, acc, deg3, W_l, b_l, W_r, b_r):
    R = 1000

    def body(x_ref, a0_ref, a1_ref, d0_ref, d1_ref, wl_ref, bl_ref, wr_ref,
             br_ref, out_ref):
        a = a0_ref[0] + a1_ref[0]
        d = d0_ref[0] + d1_ref[0]
        mean = a / jnp.maximum(d, 1.0)
        h = (jnp.dot(mean, wl_ref[...], preferred_element_type=jnp.float32)
             + jnp.dot(x_ref[...], wr_ref[...], preferred_element_type=jnp.float32)
             + bl_ref[...] + br_ref[...])
        norm = jnp.sqrt(jnp.sum(h * h, axis=1, keepdims=True))
        out_ref[...] = jnp.maximum(h / jnp.maximum(norm, 1e-12), 0.0)

    return pl.pallas_call(
        body,
        grid=(N_NODES // R,),
        in_specs=[
            pl.BlockSpec((R, D), lambda i: (i, 0)),
            pl.BlockSpec((1, R, D), lambda i: (0, i, 0)),
            pl.BlockSpec((1, R, D), lambda i: (1, i, 0)),
            pl.BlockSpec((1, R, 1), lambda i: (0, i, 0)),
            pl.BlockSpec((1, R, 1), lambda i: (1, i, 0)),
            pl.BlockSpec((D, D), lambda i: (0, 0)),
            pl.BlockSpec((1, D), lambda i: (0, 0)),
            pl.BlockSpec((D, D), lambda i: (0, 0)),
            pl.BlockSpec((1, D), lambda i: (0, 0)),
        ],
        out_specs=pl.BlockSpec((R, D), lambda i: (i, 0)),
        out_shape=jax.ShapeDtypeStruct((N_NODES, D), jnp.float32),
    )(x, acc, acc, deg3, deg3, W_l, b_l.reshape(1, D), W_r, b_r.reshape(1, D))


def kernel(x, edge_index, W_l, b_l, W_r, b_r):
    ei = edge_index.astype(jnp.int32).reshape(-1)
    z2 = jnp.zeros((CHUNK, D), jnp.float32)
    z1 = jnp.zeros((N_PAD,), jnp.float32)
    acc, deg = _sc_aggregate(x, ei, z2, z1)
    return _tc_epilogue(x, acc, deg[..., None], W_l, b_l, W_r, b_r)


# speedup vs baseline: 1.1358x; 1.1358x over previous
import functools

import jax
import jax.numpy as jnp
from jax import lax
from jax.experimental import pallas as pl
from jax.experimental.pallas import tpu as pltpu
from jax.experimental.pallas import tpu_sc as plsc

N_NODES = 10000
N_EDGES = 320000
D = 128

NC = 2
NS = 16
NW = NC * NS
N_PAD = 10240
ROWS_PER_TILE = N_PAD // NS
CHUNK = 128
NCHUNK = 79
EPW = NCHUNK * CHUNK
E_PAD = NW * EPW
DST_BITS = 14


def _sc_aggregate(x, packed3, z2, z1):
    mesh = plsc.VectorSubcoreMesh(core_axis_name="c", subcore_axis_name="s")

    @functools.partial(
        pl.kernel,
        out_type=[
            jax.ShapeDtypeStruct((NC, N_PAD, D), jnp.float32),
            jax.ShapeDtypeStruct((NC, N_PAD), jnp.float32),
        ],
        mesh=mesh,
        scratch_types=[
            pltpu.VMEM((NCHUNK, CHUNK), jnp.int32),
            pltpu.VMEM((CHUNK,), jnp.int32),
            pltpu.VMEM((CHUNK,), jnp.int32),
            pltpu.VMEM((CHUNK,), jnp.int32),
            pltpu.VMEM((CHUNK,), jnp.int32),
            pltpu.VMEM((CHUNK, D), jnp.float32),
            pltpu.VMEM((CHUNK, D), jnp.float32),
            pltpu.VMEM((CHUNK,), jnp.float32),
            pltpu.VMEM_SHARED((N_PAD, D), jnp.float32),
            pltpu.VMEM_SHARED((N_PAD,), jnp.float32),
            pltpu.SemaphoreType.DMA,
            pltpu.SemaphoreType.DMA,
        ],
    )
    def agg(x_hbm, pk_hbm, z2_hbm, z1_hbm, acc_out, deg_out,
            pk_t, src_a, src_b, dst_a, dst_b, rows_a, rows_b, ones_v,
            acc_s, deg_s, sem_a, sem_b):
        c = lax.axis_index("c")
        s = lax.axis_index("s")
        wid = s * NC + c
        rbase = s * ROWS_PER_TILE

        pltpu.sync_copy(pk_hbm.at[wid], pk_t)

        pltpu.sync_copy(z2_hbm, rows_a)
        for j in range(ROWS_PER_TILE // CHUNK):
            pltpu.sync_copy(rows_a, acc_s.at[pl.ds(rbase + j * CHUNK, CHUNK)])
        pltpu.sync_copy(z1_hbm.at[pl.ds(rbase, ROWS_PER_TILE)],
                        deg_s.at[pl.ds(rbase, ROWS_PER_TILE)])
        for j in range(CHUNK // 16):
            ones_v[pl.ds(j * 16, 16)] = jnp.ones((16,), jnp.float32)
        plsc.subcore_barrier()

        mask = jnp.int32((1 << DST_BITS) - 1)

        def unpack(k, src_v, dst_v):
            for j in range(CHUNK // 16):
                p = pk_t[k, pl.ds(j * 16, 16)]
                src_v[pl.ds(j * 16, 16)] = lax.shift_right_logical(
                    p, DST_BITS)
                dst_v[pl.ds(j * 16, 16)] = lax.bitwise_and(p, mask)

        def gather(src_v, buf, sem):
            pltpu.async_copy(x_hbm.at[src_v], buf, sem)

        def wait(src_v, buf, sem):
            pltpu.make_async_copy(x_hbm.at[src_v], buf, sem).wait()

        def flush(dst_v, buf):
            pltpu.sync_copy(buf, acc_s.at[dst_v], add=True)
            pltpu.sync_copy(ones_v, deg_s.at[dst_v], add=True)

        unpack(0, src_a, dst_a)
        gather(src_a, rows_a, sem_a)

        def body(k2, carry):
            k = 2 * k2
            unpack(k + 1, src_b, dst_b)
            gather(src_b, rows_b, sem_b)
            wait(src_a, rows_a, sem_a)
            flush(dst_a, rows_a)
            unpack(k + 2, src_a, dst_a)
            gather(src_a, rows_a, sem_a)
            wait(src_b, rows_b, sem_b)
            flush(dst_b, rows_b)
            return carry

        lax.fori_loop(0, (NCHUNK - 1) // 2, body, 0)
        wait(src_a, rows_a, sem_a)
        flush(dst_a, rows_a)
        plsc.subcore_barrier()

        pltpu.sync_copy(acc_s.at[pl.ds(rbase, ROWS_PER_TILE)],
                        acc_out.at[c, pl.ds(rbase, ROWS_PER_TILE)])
        pltpu.sync_copy(deg_s.at[pl.ds(rbase, ROWS_PER_TILE)],
                        deg_out.at[c, pl.ds(rbase, ROWS_PER_TILE)])

    return agg(x, packed3, z2, z1)


def _tc_xr(x, W_r, b_l, b_r):
    R = 1000

    def body(x_ref, wr_ref, bl_ref, br_ref, out_ref):
        out_ref[...] = (jnp.dot(x_ref[...], wr_ref[...],
                                preferred_element_type=jnp.float32)
                        + bl_ref[...] + br_ref[...])

    return pl.pallas_call(
        body,
        grid=(N_NODES // R,),
        in_specs=[
            pl.BlockSpec((R, D), lambda i: (i, 0)),
            pl.BlockSpec((D, D), lambda i: (0, 0)),
            pl.BlockSpec((1, D), lambda i: (0, 0)),
            pl.BlockSpec((1, D), lambda i: (0, 0)),
        ],
        out_specs=pl.BlockSpec((R, D), lambda i: (i, 0)),
        out_shape=jax.ShapeDtypeStruct((N_NODES, D), jnp.float32),
    )(x, W_r, b_l.reshape(1, D), b_r.reshape(1, D))


def _tc_epilogue(xr, acc, deg3, W_l):
    R = 1000

    def body(xr_ref, a0_ref, a1_ref, d0_ref, d1_ref, wl_ref, out_ref):
        a = a0_ref[0] + a1_ref[0]
        d = d0_ref[0] + d1_ref[0]
        mean = a / jnp.maximum(d, 1.0)
        h = (jnp.dot(mean, wl_ref[...], preferred_element_type=jnp.float32)
             + xr_ref[...])
        norm = jnp.sqrt(jnp.sum(h * h, axis=1, keepdims=True))
        out_ref[...] = jnp.maximum(h / jnp.maximum(norm, 1e-12), 0.0)

    return pl.pallas_call(
        body,
        grid=(N_NODES // R,),
        in_specs=[
            pl.BlockSpec((R, D), lambda i: (i, 0)),
            pl.BlockSpec((1, R, D), lambda i: (0, i, 0)),
            pl.BlockSpec((1, R, D), lambda i: (1, i, 0)),
            pl.BlockSpec((1, R, 1), lambda i: (0, i, 0)),
            pl.BlockSpec((1, R, 1), lambda i: (1, i, 0)),
            pl.BlockSpec((D, D), lambda i: (0, 0)),
        ],
        out_specs=pl.BlockSpec((R, D), lambda i: (i, 0)),
        out_shape=jax.ShapeDtypeStruct((N_NODES, D), jnp.float32),
    )(xr, acc, acc, deg3, deg3, W_l)


def kernel(x, edge_index, W_l, b_l, W_r, b_r):
    src = edge_index[0].astype(jnp.int32)
    dst = edge_index[1].astype(jnp.int32)
    npad = E_PAD - N_EDGES
    pad_src = jnp.arange(npad, dtype=jnp.int32) % N_NODES
    pad_dst = jnp.arange(npad, dtype=jnp.int32) % (N_PAD - N_NODES) + N_NODES
    src_p = jnp.concatenate([src, pad_src])
    dst_p = jnp.concatenate([dst, pad_dst])
    packed3 = ((src_p << DST_BITS) | dst_p).reshape(NW, NCHUNK, CHUNK)
    z2 = jnp.zeros((CHUNK, D), jnp.float32)
    z1 = jnp.zeros((N_PAD,), jnp.float32)
    acc, deg = _sc_aggregate(x, packed3, z2, z1)
    xr = _tc_xr(x, W_r, b_l, b_r)
    return _tc_epilogue(xr, acc, deg[..., None], W_l)


# speedup vs baseline: 1.1569x; 1.0186x over previous
import functools

import jax
import jax.numpy as jnp
from jax import lax
from jax.experimental import pallas as pl
from jax.experimental.pallas import tpu as pltpu
from jax.experimental.pallas import tpu_sc as plsc

N_NODES = 10000
N_EDGES = 320000
D = 128

NC = 2
NS = 16
NW = NC * NS
N_PAD = 10240
ROWS_PER_TILE = N_PAD // NS
CHUNK = 128
NCHUNK = 79
EPW = NCHUNK * CHUNK
E_PAD = NW * EPW
DST_BITS = 14


def _sc_aggregate(x, packed3, z2, z1):
    mesh = plsc.VectorSubcoreMesh(core_axis_name="c", subcore_axis_name="s")

    @functools.partial(
        pl.kernel,
        out_type=[
            jax.ShapeDtypeStruct((NC, N_PAD, D), jnp.float32),
            jax.ShapeDtypeStruct((NC, N_PAD), jnp.float32),
        ],
        mesh=mesh,
        compiler_params=pltpu.CompilerParams(use_tc_tiling_on_sc=True),
        scratch_types=[
            pltpu.VMEM((NCHUNK, CHUNK), jnp.int32),
            pltpu.VMEM((CHUNK,), jnp.int32),
            pltpu.VMEM((CHUNK,), jnp.int32),
            pltpu.VMEM((CHUNK,), jnp.int32),
            pltpu.VMEM((CHUNK,), jnp.int32),
            pltpu.VMEM((CHUNK, D), jnp.float32),
            pltpu.VMEM((CHUNK, D), jnp.float32),
            pltpu.VMEM((CHUNK,), jnp.float32),
            pltpu.VMEM_SHARED((N_PAD, D), jnp.float32),
            pltpu.VMEM_SHARED((N_PAD,), jnp.float32),
            pltpu.SemaphoreType.DMA,
            pltpu.SemaphoreType.DMA,
        ],
    )
    def agg(x_hbm, pk_hbm, z2_hbm, z1_hbm, acc_out, deg_out,
            pk_t, src_a, src_b, dst_a, dst_b, rows_a, rows_b, ones_v,
            acc_s, deg_s, sem_a, sem_b):
        c = lax.axis_index("c")
        s = lax.axis_index("s")
        wid = s * NC + c
        rbase = s * ROWS_PER_TILE

        pltpu.sync_copy(pk_hbm.at[wid], pk_t)

        pltpu.sync_copy(z2_hbm, rows_a)
        for j in range(ROWS_PER_TILE // CHUNK):
            pltpu.sync_copy(rows_a, acc_s.at[pl.ds(rbase + j * CHUNK, CHUNK)])
        pltpu.sync_copy(z1_hbm.at[pl.ds(rbase, ROWS_PER_TILE)],
                        deg_s.at[pl.ds(rbase, ROWS_PER_TILE)])
        for j in range(CHUNK // 16):
            ones_v[pl.ds(j * 16, 16)] = jnp.ones((16,), jnp.float32)
        plsc.subcore_barrier()

        mask = jnp.int32((1 << DST_BITS) - 1)

        def unpack(k, src_v, dst_v):
            for j in range(CHUNK // 16):
                p = pk_t[k, pl.ds(j * 16, 16)]
                src_v[pl.ds(j * 16, 16)] = lax.shift_right_logical(
                    p, DST_BITS)
                dst_v[pl.ds(j * 16, 16)] = lax.bitwise_and(p, mask)

        def gather(src_v, buf, sem):
            pltpu.async_copy(x_hbm.at[src_v], buf, sem)

        def wait(src_v, buf, sem):
            pltpu.make_async_copy(x_hbm.at[src_v], buf, sem).wait()

        def flush(dst_v, buf):
            pltpu.sync_copy(buf, acc_s.at[dst_v], add=True)
            pltpu.sync_copy(ones_v, deg_s.at[dst_v], add=True)

        unpack(0, src_a, dst_a)
        gather(src_a, rows_a, sem_a)

        def body(k2, carry):
            k = 2 * k2
            unpack(k + 1, src_b, dst_b)
            gather(src_b, rows_b, sem_b)
            wait(src_a, rows_a, sem_a)
            flush(dst_a, rows_a)
            unpack(k + 2, src_a, dst_a)
            gather(src_a, rows_a, sem_a)
            wait(src_b, rows_b, sem_b)
            flush(dst_b, rows_b)
            return carry

        lax.fori_loop(0, (NCHUNK - 1) // 2, body, 0)
        wait(src_a, rows_a, sem_a)
        flush(dst_a, rows_a)
        plsc.subcore_barrier()

        pltpu.sync_copy(acc_s.at[pl.ds(rbase, ROWS_PER_TILE)],
                        acc_out.at[c, pl.ds(rbase, ROWS_PER_TILE)])
        pltpu.sync_copy(deg_s.at[pl.ds(rbase, ROWS_PER_TILE)],
                        deg_out.at[c, pl.ds(rbase, ROWS_PER_TILE)])

    return agg(x, packed3, z2, z1)


def _tc_epilogue(x, acc, deg3, W_l, b_l, W_r, b_r):
    R = 2000

    def body(x_ref, a0_ref, a1_ref, d0_ref, d1_ref, wl_ref, bl_ref, wr_ref,
             br_ref, out_ref):
        a = a0_ref[0] + a1_ref[0]
        d = d0_ref[0] + d1_ref[0]
        mean = a / jnp.maximum(d, 1.0)
        h = (jnp.dot(mean, wl_ref[...], preferred_element_type=jnp.float32)
             + jnp.dot(x_ref[...], wr_ref[...], preferred_element_type=jnp.float32)
             + bl_ref[...] + br_ref[...])
        norm = jnp.sqrt(jnp.sum(h * h, axis=1, keepdims=True))
        out_ref[...] = jnp.maximum(h / jnp.maximum(norm, 1e-12), 0.0)

    return pl.pallas_call(
        body,
        grid=(N_NODES // R,),
        in_specs=[
            pl.BlockSpec((R, D), lambda i: (i, 0)),
            pl.BlockSpec((1, R, D), lambda i: (0, i, 0)),
            pl.BlockSpec((1, R, D), lambda i: (1, i, 0)),
            pl.BlockSpec((1, R, 1), lambda i: (0, i, 0)),
            pl.BlockSpec((1, R, 1), lambda i: (1, i, 0)),
            pl.BlockSpec((D, D), lambda i: (0, 0)),
            pl.BlockSpec((1, D), lambda i: (0, 0)),
            pl.BlockSpec((D, D), lambda i: (0, 0)),
            pl.BlockSpec((1, D), lambda i: (0, 0)),
        ],
        out_specs=pl.BlockSpec((R, D), lambda i: (i, 0)),
        out_shape=jax.ShapeDtypeStruct((N_NODES, D), jnp.float32),
    )(x, acc, acc, deg3, deg3, W_l, b_l.reshape(1, D), W_r, b_r.reshape(1, D))


def kernel(x, edge_index, W_l, b_l, W_r, b_r):
    src = edge_index[0].astype(jnp.int32)
    dst = edge_index[1].astype(jnp.int32)
    npad = E_PAD - N_EDGES
    pad_src = jnp.arange(npad, dtype=jnp.int32) % N_NODES
    pad_dst = jnp.arange(npad, dtype=jnp.int32) % (N_PAD - N_NODES) + N_NODES
    src_p = jnp.concatenate([src, pad_src])
    dst_p = jnp.concatenate([dst, pad_dst])
    packed3 = ((src_p << DST_BITS) | dst_p).reshape(NW, NCHUNK, CHUNK)
    z2 = jnp.zeros((CHUNK, D), jnp.float32)
    z1 = jnp.zeros((N_PAD,), jnp.float32)
    acc, deg = _sc_aggregate(x, packed3, z2, z1)
    return _tc_epilogue(x, acc, deg[..., None], W_l, b_l, W_r, b_r)
